# baseline (speedup 1.0000x reference)
import jax
import jax.numpy as jnp
from jax.experimental import pallas as pl
from jax.experimental.pallas import tpu as pltpu

BS = 128
NU = BS * 300
NS = BS * 30
HC = 128
H = 2
DH = 64
L = 4


def _uhead_body(x_ref, w1_ref, b1_ref, w2_ref, b2_ref, o_ref):
    h = jnp.maximum(x_ref[...] @ w1_ref[...] + b1_ref[...], 0.0)
    o_ref[...] = jnp.tanh(h @ w2_ref[...] + b2_ref[...]) * 10.0


def _uhead(x, fuW1, fub1, fuW2, fub2):
    blk = 1536
    w2p = jnp.zeros((HC, 128), jnp.float32).at[:, 0].set(fuW2[:, 0])
    b2p = jnp.zeros((1, 128), jnp.float32).at[0, 0].set(fub2[0])
    out = pl.pallas_call(
        _uhead_body,
        grid=(NU // blk,),
        in_specs=[
            pl.BlockSpec((blk, HC), lambda i: (i, 0)),
            pl.BlockSpec((HC, HC), lambda i: (0, 0)),
            pl.BlockSpec((1, HC), lambda i: (0, 0)),
            pl.BlockSpec((HC, 128), lambda i: (0, 0)),
            pl.BlockSpec((1, 128), lambda i: (0, 0)),
        ],
        out_specs=pl.BlockSpec((blk, 128), lambda i: (i, 0)),
        out_shape=jax.ShapeDtypeStruct((NU, 128), jnp.float32),
    )(x, fuW1, fub1.reshape(1, HC), w2p, b2p)
    return out[:, 0]


def _combine(Wx, Wrel):
    Wh = Wx.reshape(HC, H, DH)
    return jnp.einsum("chd,hdf->chf", Wh, Wrel).reshape(HC, HC)


def kernel(x_units, x_src, x_dst, mask1, mask2, e_in_u, e_in_s, e_con_s,
           e_con_u, e_mov_u, e_mov_d, e_dcon_d, e_dcon_u, e_corr_d,
           e_corr_s, e_rcorr_s, e_rcorr_d, W_in, b_in, Wk, Wq, Wv, Wo,
           Wrel_att, Wrel_msg, skip, Wout, bout, fuW1, fub1, fuW2, fub2,
           fsW1, fsb1, fsW2, fsb2):
    Ns = [NU, NS, NS]
    xs = [jax.nn.relu(x @ W_in[t] + b_in[t])
          for t, x in enumerate([x_units, x_src, x_dst])]
    rels = [(0, 1, e_in_u, e_in_s), (1, 0, e_con_s, e_con_u),
            (0, 2, e_mov_u, e_mov_d), (2, 0, e_dcon_d, e_dcon_u),
            (2, 1, e_corr_d, e_corr_s), (1, 2, e_rcorr_s, e_rcorr_d)]
    for l in range(L):
        q_t = [(xs[t] @ Wq[l, t]).reshape(-1, H, DH) for t in range(3)]
        agg = [jnp.zeros((Ns[t], H, DH), jnp.float32) for t in range(3)]
        for r, (st, dt, sidx, didx) in enumerate(rels):
            kk_n = (xs[st] @ _combine(Wk[l, st], Wrel_att[l, r])).reshape(-1, H, DH)
            vv_n = (xs[st] @ _combine(Wv[l, st], Wrel_msg[l, r])).reshape(-1, H, DH)
            kk = kk_n[sidx]
            qq = q_t[dt][didx]
            score = (qq * kk).sum(-1) / jnp.sqrt(float(DH))
            smax = jax.ops.segment_max(score, didx, num_segments=Ns[dt])
            smax = jnp.where(jnp.isfinite(smax), smax, 0.0)
            ex = jnp.exp(score - smax[didx])
            den = jax.ops.segment_sum(ex, didx, num_segments=Ns[dt])
            vv = vv_n[sidx]
            unn = jax.ops.segment_sum(ex[:, :, None] * vv, didx,
                                      num_segments=Ns[dt])
            agg[dt] = agg[dt] + unn / (den[:, :, None] + 1e-16)
        new_xs = []
        for t in range(3):
            o = jax.nn.gelu(agg[t].reshape(Ns[t], HC) @ Wo[l, t])
            beta = jax.nn.sigmoid(skip[l, t])
            new_xs.append(beta * o + (1.0 - beta) * xs[t])
        xs = new_xs
    xs = [xs[t] @ Wout[t] + bout[t] for t in range(3)]
    u = _uhead(xs[0], fuW1, fub1, fuW2, fub2)
    logits1 = u.reshape(BS, 300) + mask1
    logp1_all = jax.nn.log_softmax(logits1, axis=-1)
    act1 = jnp.argmax(logits1, axis=-1)
    logp1 = jnp.take_along_axis(logp1_all, act1[:, None], axis=1)[:, 0]
    p1 = jax.nn.softmax(logits1, axis=-1)
    ent1 = -(p1 * logp1_all).sum(-1)
    units_dense = xs[0].reshape(BS, 300, HC)
    servers_dense = xs[1].reshape(BS, 30, HC)
    unit_feat = jnp.take_along_axis(units_dense, act1[:, None, None], axis=1)
    combined = jnp.concatenate(
        [servers_dense, jnp.broadcast_to(unit_feat, (BS, 30, HC))], axis=-1)
    s2 = jnp.tanh(jax.nn.relu(combined @ fsW1 + fsb1) @ fsW2 + fsb2)[..., 0] * 10.0
    logits2 = s2 + mask2
    logp2_all = jax.nn.log_softmax(logits2, axis=-1)
    act2 = jnp.argmax(logits2, axis=-1)
    logp2 = jnp.take_along_axis(logp2_all, act2[:, None], axis=1)[:, 0]
    p2 = jax.nn.softmax(logits2, axis=-1)
    ent2 = -(p2 * logp2_all).sum(-1)
    return jnp.stack([logp1, logp2, ent1 + ent2], axis=-1)


# speedup vs baseline: 31.8684x; 31.8684x over previous
import functools

import jax
import jax.numpy as jnp
from jax import lax
from jax.experimental import pallas as pl
from jax.experimental.pallas import tpu as pltpu
from jax.experimental.pallas import tpu_sc as plsc

BS = 128
NU = BS * 300
NS = BS * 30
HC = 128
H = 2
DH = 64
L = 4

NC, NSUB = 2, 16
NW = NC * NSUB
C = 128
AGGW = 144
EPAD = 256

RELS = [(0, 1, NU), (1, 0, NU), (0, 2, NU * 8), (2, 0, NU * 8),
        (2, 1, NS), (1, 2, NS)]
NNODES = [NU, NS, NS]
RCFG = {0: (1, 120, 2048), 1: (6, 200, 512), 2: (1, 120, 12288),
        3: (6, 200, 2304), 4: (1, 120, 512), 5: (1, 120, 512)}


@functools.lru_cache(None)
def _lin_relu(N, blk):
    def body(x_ref, w_ref, b_ref, o_ref):
        o_ref[...] = jnp.maximum(
            jnp.dot(x_ref[...], w_ref[...],
                    preferred_element_type=jnp.float32) + b_ref[...], 0.0)
    return pl.pallas_call(
        body, grid=(N // blk,),
        in_specs=[pl.BlockSpec((blk, HC), lambda i: (i, 0)),
                  pl.BlockSpec((HC, HC), lambda i: (0, 0)),
                  pl.BlockSpec((1, HC), lambda i: (0, 0))],
        out_specs=pl.BlockSpec((blk, HC), lambda i: (i, 0)),
        out_shape=jax.ShapeDtypeStruct((N, HC), jnp.float32))


@functools.lru_cache(None)
def _mmulti(N, nw, blk):
    def body(*refs):
        x = refs[0][...]
        for i in range(nw):
            refs[1 + nw + i][...] = jnp.dot(
                x, refs[1 + i][...], preferred_element_type=jnp.float32)
    return pl.pallas_call(
        body, grid=(N // blk,),
        in_specs=[pl.BlockSpec((blk, HC), lambda i: (i, 0))] +
                 [pl.BlockSpec((HC, HC), lambda i: (0, 0))] * nw,
        out_specs=[pl.BlockSpec((blk, HC), lambda i: (i, 0))] * nw,
        out_shape=[jax.ShapeDtypeStruct((N, HC), jnp.float32)] * nw)


@functools.lru_cache(None)
def _outmix(N, blk):
    def body(a1_ref, a2_ref, x_ref, wo_ref, beta_ref, o_ref):
        s = a1_ref[...] + a2_ref[...]
        mm = jnp.dot(s, wo_ref[...], preferred_element_type=jnp.float32)
        g = 0.5 * mm * (1.0 + jnp.tanh(
            0.7978845608028654 * (mm + 0.044715 * mm * mm * mm)))
        b = beta_ref[...]
        o_ref[...] = b * g + (1.0 - b) * x_ref[...]
    return pl.pallas_call(
        body, grid=(N // blk,),
        in_specs=[pl.BlockSpec((blk, HC), lambda i: (i, 0)),
                  pl.BlockSpec((blk, HC), lambda i: (i, 0)),
                  pl.BlockSpec((blk, HC), lambda i: (i, 0)),
                  pl.BlockSpec((HC, HC), lambda i: (0, 0)),
                  pl.BlockSpec((1, HC), lambda i: (0, 0))],
        out_specs=pl.BlockSpec((blk, HC), lambda i: (i, 0)),
        out_shape=jax.ShapeDtypeStruct((N, HC), jnp.float32))


@functools.lru_cache(None)
def _outproj_units(blk):
    def body(x_ref, wout_ref, bout_ref, w1_ref, b1_ref, w2_ref, b2_ref,
             y_ref, u_ref):
        y = jnp.dot(x_ref[...], wout_ref[...],
                    preferred_element_type=jnp.float32) + bout_ref[...]
        y_ref[...] = y
        h = jnp.maximum(jnp.dot(y, w1_ref[...],
                                preferred_element_type=jnp.float32)
                        + b1_ref[...], 0.0)
        u_ref[...] = jnp.tanh(
            jnp.dot(h, w2_ref[...], preferred_element_type=jnp.float32)
            + b2_ref[...]) * 10.0
    return pl.pallas_call(
        body, grid=(NU // blk,),
        in_specs=[pl.BlockSpec((blk, HC), lambda i: (i, 0))] +
                 [pl.BlockSpec((HC, HC), lambda i: (0, 0)),
                  pl.BlockSpec((1, HC), lambda i: (0, 0)),
                  pl.BlockSpec((HC, HC), lambda i: (0, 0)),
                  pl.BlockSpec((1, HC), lambda i: (0, 0)),
                  pl.BlockSpec((HC, HC), lambda i: (0, 0)),
                  pl.BlockSpec((1, HC), lambda i: (0, 0))],
        out_specs=[pl.BlockSpec((blk, HC), lambda i: (i, 0)),
                   pl.BlockSpec((blk, HC), lambda i: (i, 0))],
        out_shape=[jax.ShapeDtypeStruct((NU, HC), jnp.float32),
                   jax.ShapeDtypeStruct((NU, HC), jnp.float32)])


@functools.lru_cache(None)
def _lin(N, blk):
    def body(x_ref, w_ref, b_ref, o_ref):
        o_ref[...] = jnp.dot(x_ref[...], w_ref[...],
                             preferred_element_type=jnp.float32) + b_ref[...]
    return pl.pallas_call(
        body, grid=(N // blk,),
        in_specs=[pl.BlockSpec((blk, HC), lambda i: (i, 0)),
                  pl.BlockSpec((HC, HC), lambda i: (0, 0)),
                  pl.BlockSpec((1, HC), lambda i: (0, 0))],
        out_specs=pl.BlockSpec((blk, HC), lambda i: (i, 0)),
        out_shape=jax.ShapeDtypeStruct((N, HC), jnp.float32))


@functools.lru_cache(None)
def _ufb():
    def body(x_ref, w_ref, b_ref, o_ref):
        o_ref[...] = jnp.dot(x_ref[...], w_ref[...],
                             preferred_element_type=jnp.float32) + b_ref[...]
    return pl.pallas_call(
        body,
        in_specs=[pl.BlockSpec((BS, HC), lambda: (0, 0)),
                  pl.BlockSpec((HC, 2 * HC), lambda: (0, 0)),
                  pl.BlockSpec((1, 2 * HC), lambda: (0, 0))],
        out_specs=pl.BlockSpec((BS, 2 * HC), lambda: (0, 0)),
        out_shape=jax.ShapeDtypeStruct((BS, 2 * HC), jnp.float32))


@functools.lru_cache(None)
def _stage2(blk):
    def body(s_ref, u_ref, a_ref, w2_ref, b2_ref, o_ref):
        h = jnp.maximum(
            jnp.dot(s_ref[...], a_ref[...],
                    preferred_element_type=jnp.float32) + u_ref[...], 0.0)
        o_ref[...] = jnp.tanh(
            jnp.dot(h, w2_ref[...], preferred_element_type=jnp.float32)
            + b2_ref[...]) * 10.0
    return pl.pallas_call(
        body, grid=(NS // blk,),
        in_specs=[pl.BlockSpec((blk, HC), lambda i: (i, 0)),
                  pl.BlockSpec((blk, 2 * HC), lambda i: (i, 0)),
                  pl.BlockSpec((HC, 2 * HC), lambda i: (0, 0)),
                  pl.BlockSpec((2 * HC, HC), lambda i: (0, 0)),
                  pl.BlockSpec((1, HC), lambda i: (0, 0))],
        out_specs=pl.BlockSpec((blk, HC), lambda i: (i, 0)),
        out_shape=jax.ShapeDtypeStruct((NS, HC), jnp.float32))


@functools.lru_cache(None)
def _head(P):
    def body(l_ref, o_ref):
        l = l_ref[...]
        m = jnp.max(l, axis=1, keepdims=True)
        e = jnp.exp(l - m)
        s = jnp.sum(e, axis=1, keepdims=True)
        logs = jnp.log(s)
        p = e / s
        lp = l - (m + logs)
        ent = -jnp.sum(p * lp, axis=1, keepdims=True)
        logp = -logs
        io = lax.broadcasted_iota(jnp.int32, (BS, P), 1).astype(jnp.float32)
        am = jnp.min(jnp.where(l >= m, io, float(P)), axis=1, keepdims=True)
        pad = jnp.zeros((BS, 125), jnp.float32)
        o_ref[...] = jnp.concatenate([logp, ent, am, pad], axis=1)
    return pl.pallas_call(
        body,
        in_specs=[pl.BlockSpec((BS, P), lambda: (0, 0))],
        out_specs=pl.BlockSpec((BS, 128), lambda: (0, 0)),
        out_shape=jax.ShapeDtypeStruct((BS, 128), jnp.float32))


@functools.lru_cache(None)
def _edge_kernel(Nsrc, Nd, E, nsub_per, rows_sub, cap):
    nsub = NW * nsub_per
    npad = ((nsub + 1 + 15) // 16) * 16
    rows_buf = rows_sub + 1
    G = C // 16
    mesh = plsc.VectorSubcoreMesh(core_axis_name="c", subcore_axis_name="s",
                                  num_cores=NC, num_subcores=NSUB)

    @functools.partial(
        pl.kernel,
        out_type=jax.ShapeDtypeStruct((Nd, HC), jnp.float32),
        mesh=mesh,
        scratch_types=[
            pltpu.VMEM((npad,), jnp.int32),
            pltpu.VMEM((C,), jnp.int32),
            pltpu.VMEM((C,), jnp.int32),
            pltpu.VMEM((C, HC), jnp.float32),
            pltpu.VMEM((C, HC), jnp.float32),
            pltpu.VMEM((cap,), jnp.float32),
            pltpu.VMEM((cap,), jnp.float32),
            pltpu.VMEM((rows_buf * AGGW,), jnp.float32),
            pltpu.VMEM((rows_sub, HC), jnp.float32),
            pltpu.SemaphoreType.DMA,
            pltpu.SemaphoreType.DMA,
        ],
        compiler_params=pltpu.CompilerParams(needs_layout_passes=False),
    )
    def kern(ktab, qtab, vtab, ss, sd, bnd, out, bndv, ssv, sdv, bufa,
             bufb, sco0, sco1, agg, outb, sem1, sem2):
        wid = lax.axis_index("s") * NC + lax.axis_index("c")
        pltpu.sync_copy(bnd, bndv)
        zero16 = jnp.zeros((16,), jnp.float32)
        lane = lax.iota(jnp.int32, 16)

        def bnd_at(pos):
            acc = zero16
            for b in range(npad // 16):
                v = bndv[pl.ds(b * 16, 16)].astype(jnp.float32)
                acc = acc + jnp.where(lane + b * 16 == pos, v, 0.0)
            return jnp.sum(acc).astype(jnp.int32)

        def sub_body(sub, _carry):
            slab = wid * nsub_per + sub
            e0 = bnd_at(slab)
            e1 = bnd_at(slab + 1)
            eb = jnp.bitwise_and(e0, jnp.int32(~7))
            nch = jnp.minimum(
                jnp.right_shift(e1 - eb + (C - 1), 7), cap // C)
            dlo = slab * rows_sub

            def zs(i, _):
                sco0[pl.ds(i * 16, 16)] = zero16
                sco1[pl.ds(i * 16, 16)] = zero16
                return 0
            lax.fori_loop(0, cap // 16, zs, 0)

            def za(i, _):
                agg[pl.ds(i * 16, 16)] = zero16
                return 0
            lax.fori_loop(0, rows_buf * (AGGW // 16), za, 0)

            def cha(ci, _):
                base = pl.multiple_of(eb + ci * C, 8)
                pltpu.sync_copy(ss.at[pl.ds(base, C)], ssv)
                pltpu.sync_copy(sd.at[pl.ds(base, C)], sdv)
                ca = pltpu.async_copy(ktab.at[ssv], bufa, sem1)
                cb = pltpu.async_copy(qtab.at[sdv], bufb, sem2)
                ca.wait()
                cb.wait()

                def grp(g, _):
                    s0v = zero16
                    s1v = zero16
                    for j in range(16):
                        e = g * 16 + j
                        a0 = bufa[e, pl.ds(0, 16)] * bufb[e, pl.ds(0, 16)]
                        a1 = bufa[e, pl.ds(64, 16)] * bufb[e, pl.ds(64, 16)]
                        for kk in range(1, 4):
                            a0 = a0 + (bufa[e, pl.ds(kk * 16, 16)]
                                       * bufb[e, pl.ds(kk * 16, 16)])
                            a1 = a1 + (bufa[e, pl.ds(64 + kk * 16, 16)]
                                       * bufb[e, pl.ds(64 + kk * 16, 16)])
                        s0v = jnp.where(lane == j, jnp.sum(a0) * 0.125, s0v)
                        s1v = jnp.where(lane == j, jnp.sum(a1) * 0.125, s1v)
                    le = ci * C + g * 16
                    sco0[pl.ds(le, 16)] = s0v
                    sco1[pl.ds(le, 16)] = s1v
                    return 0
                lax.fori_loop(0, G, grp, 0)
                return 0
            lax.fori_loop(0, nch, cha, 0)

            nsc = nch * G

            def mx(i, m):
                m = jnp.maximum(m, jnp.max(sco0[pl.ds(i * 16, 16)]))
                return jnp.maximum(m, jnp.max(sco1[pl.ds(i * 16, 16)]))
            m = lax.fori_loop(0, nsc, mx, jnp.float32(-1e30))

            def exs(i, _):
                sco0[pl.ds(i * 16, 16)] = jnp.exp(sco0[pl.ds(i * 16, 16)] - m)
                sco1[pl.ds(i * 16, 16)] = jnp.exp(sco1[pl.ds(i * 16, 16)] - m)
                return 0
            lax.fori_loop(0, nsc, exs, 0)

            def chb(ci, _):
                base = pl.multiple_of(eb + ci * C, 8)
                pltpu.sync_copy(ss.at[pl.ds(base, C)], ssv)
                pltpu.sync_copy(sd.at[pl.ds(base, C)], sdv)
                pltpu.async_copy(vtab.at[ssv], bufa, sem1).wait()

                def grp(g, _):
                    le = ci * C + g * 16
                    exv0 = sco0[pl.ds(le, 16)]
                    exv1 = sco1[pl.ds(le, 16)]
                    sdl = sdv[pl.ds(g * 16, 16)]
                    for j in range(16):
                        e = g * 16 + j
                        lr = sdl[j] - dlo
                        ok = jnp.logical_and(lr >= 0, lr < rows_sub)
                        row = jnp.where(ok, lr, rows_sub)
                        rb = pl.multiple_of(row * AGGW, 16)
                        ex0 = exv0[j]
                        ex1 = exv1[j]
                        for kk in range(4):
                            plsc.addupdate(
                                agg.at[pl.ds(rb + kk * 16, 16)],
                                bufa[e, pl.ds(kk * 16, 16)] * ex0)
                        for kk in range(4, 8):
                            plsc.addupdate(
                                agg.at[pl.ds(rb + kk * 16, 16)],
                                bufa[e, pl.ds(kk * 16, 16)] * ex1)
                        dv = jnp.where(lane == 0, ex0,
                                       jnp.where(lane == 1, ex1, 0.0))
                        plsc.addupdate(agg.at[pl.ds(rb + 128, 16)], dv)
                    return 0
                lax.fori_loop(0, G, grp, 0)
                return 0
            lax.fori_loop(0, nch, chb, 0)

            def fl(r, _):
                rb = pl.multiple_of(r * AGGW, 16)
                den = agg[pl.ds(rb + 128, 16)]
                d0 = zero16 + den[0] + 1e-16
                d1 = zero16 + den[1] + 1e-16
                for kk in range(4):
                    outb[r, pl.ds(kk * 16, 16)] = (
                        agg[pl.ds(rb + kk * 16, 16)] / d0)
                for kk in range(4, 8):
                    outb[r, pl.ds(kk * 16, 16)] = (
                        agg[pl.ds(rb + kk * 16, 16)] / d1)
                return 0
            lax.fori_loop(0, rows_sub, fl, 0)
            pltpu.sync_copy(outb, out.at[pl.ds(dlo, rows_sub)])
            return 0
        lax.fori_loop(0, nsub_per, sub_body, 0)
    return kern


def _combine(Wx, Wrel):
    Wh = Wx.reshape(HC, H, DH)
    return jnp.einsum("chd,hdf->chf", Wh, Wrel).reshape(HC, HC)


def _prep_rel(sidx, didx, Nd, nsub_per):
    nsub = NW * nsub_per
    rows_sub = Nd // nsub
    di = didx.astype(jnp.int32)
    si = sidx.astype(jnp.int32)
    sd, ss = lax.sort((di, si), num_keys=1)
    tg = jnp.arange(nsub + 1, dtype=jnp.int32) * rows_sub
    bnd = jnp.searchsorted(sd, tg, side="left").astype(jnp.int32)
    npad = ((nsub + 1 + 15) // 16) * 16
    bndp = jnp.zeros((npad,), jnp.int32).at[: nsub + 1].set(bnd)
    z = jnp.zeros((EPAD,), jnp.int32)
    return (jnp.concatenate([ss, z]), jnp.concatenate([sd, z]), bndp)


def kernel(x_units, x_src, x_dst, mask1, mask2, e_in_u, e_in_s, e_con_s,
           e_con_u, e_mov_u, e_mov_d, e_dcon_d, e_dcon_u, e_corr_d,
           e_corr_s, e_rcorr_s, e_rcorr_d, W_in, b_in, Wk, Wq, Wv, Wo,
           Wrel_att, Wrel_msg, skip, Wout, bout, fuW1, fub1, fuW2, fub2,
           fsW1, fsb1, fsW2, fsb2):
    blks = [1536, 768, 768]
    xs = [_lin_relu(NNODES[t], blks[t])(x, W_in[t], b_in[t].reshape(1, HC))
          for t, x in enumerate([x_units, x_src, x_dst])]

    eidx = [(e_in_u, e_in_s), (e_con_s, e_con_u), (e_mov_u, e_mov_d),
            (e_dcon_d, e_dcon_u), (e_corr_d, e_corr_s),
            (e_rcorr_s, e_rcorr_d)]
    preps = []
    for r, (st, dt, E) in enumerate(RELS):
        nsub_per, rows_sub, cap = RCFG[r]
        preps.append(_prep_rel(eidx[r][0], eidx[r][1], NNODES[dt], nsub_per))

    for l in range(L):
        active = [r for r in range(6) if not (l == L - 1 and r in (2, 5))]
        mats, keys = {t: [] for t in range(3)}, {t: [] for t in range(3)}
        for t in range(3):
            if not (l == L - 1 and t == 2):
                mats[t].append(Wq[l, t])
                keys[t].append(("q", t))
        for r in active:
            st, dt, E = RELS[r]
            mats[st].append(_combine(Wk[l, st], Wrel_att[l, r]))
            keys[st].append(("k", r))
            mats[st].append(_combine(Wv[l, st], Wrel_msg[l, r]))
            keys[st].append(("v", r))
        proj = {}
        for t in range(3):
            outs = _mmulti(NNODES[t], len(mats[t]), blks[t])(xs[t], *mats[t])
            for kkey, o in zip(keys[t], outs):
                proj[kkey] = o
        agg_by_dt = {0: [], 1: [], 2: []}
        for r in active:
            st, dt, E = RELS[r]
            nsub_per, rows_sub, cap = RCFG[r]
            ek = _edge_kernel(NNODES[st], NNODES[dt], E, nsub_per,
                              rows_sub, cap)
            ss, sd, bnd = preps[r]
            aggr = ek(proj[("k", r)], proj[("q", dt)], proj[("v", r)],
                      ss, sd, bnd)
            agg_by_dt[dt].append(aggr)
        for t in range(3):
            if l == L - 1 and t == 2:
                continue
            beta = jnp.broadcast_to(jax.nn.sigmoid(skip[l, t]), (1, HC))
            a1, a2 = agg_by_dt[t]
            xs[t] = _outmix(NNODES[t], blks[t])(a1, a2, xs[t], Wo[l, t], beta)

    w2p = jnp.zeros((HC, HC), jnp.float32).at[:, 0].set(fuW2[:, 0])
    b2p = jnp.zeros((1, HC), jnp.float32).at[0, 0].set(fub2[0])
    y0, u = _outproj_units(1536)(
        xs[0], Wout[0], bout[0].reshape(1, HC), fuW1, fub1.reshape(1, HC),
        w2p, b2p)
    y1 = _lin(NS, 768)(xs[1], Wout[1], bout[1].reshape(1, HC))

    neg = jnp.float32(-1e30)
    logits1 = u[:, 0].reshape(BS, 300) + mask1
    l1p = jnp.pad(logits1, ((0, 0), (0, 84)), constant_values=-1e30)
    h1 = _head(384)(l1p)
    logp1, ent1 = h1[:, 0], h1[:, 1]
    act1 = h1[:, 2].astype(jnp.int32)

    unit_feat = jnp.take_along_axis(
        y0.reshape(BS, 300, HC), act1[:, None, None], axis=1)[:, 0]
    ufb = _ufb()(unit_feat, fsW1[HC:], fsb1.reshape(1, 2 * HC))
    ufbx = jnp.broadcast_to(ufb[:, None, :], (BS, 30, 2 * HC)).reshape(
        NS, 2 * HC)
    w2p2 = jnp.zeros((2 * HC, HC), jnp.float32).at[:, 0].set(fsW2[:, 0])
    b2p2 = jnp.zeros((1, HC), jnp.float32).at[0, 0].set(fsb2[0])
    s2 = _stage2(768)(y1, ufbx, fsW1[:HC], w2p2, b2p2)
    logits2 = s2[:, 0].reshape(BS, 30) + mask2
    l2p = jnp.pad(logits2, ((0, 0), (0, 98)), constant_values=-1e30)
    h2 = _head(128)(l2p)
    logp2, ent2 = h2[:, 0], h2[:, 1]
    return jnp.stack([logp1, logp2, ent1 + ent2], axis=-1)
